# baseline (speedup 1.0000x reference)
import jax
import jax.numpy as jnp
from jax import lax
from jax.experimental import pallas as pl
from jax.experimental.pallas import tpu as pltpu
from jax.experimental.pallas import tpu_sc as plsc

D_MODEL = 1024
SEQ = 4
PERIOD = 8
ROWS_PER_BLOCK = 256
_CHUNK = 256
_NLANE = 16


def _sc_table_body(e2_hbm, e4_hbm, e8_hbm, out_hbm, a_v, b_v, c_v, o_v):
    c = lax.axis_index("c")
    s = lax.axis_index("s")
    w = s * 2 + c
    j = w % PERIOD
    off = (w // PERIOD) * _CHUNK
    pltpu.sync_copy(e2_hbm.at[j % 2, pl.ds(off, _CHUNK)], a_v)
    pltpu.sync_copy(e4_hbm.at[j % 4, pl.ds(off, _CHUNK)], b_v)
    pltpu.sync_copy(e8_hbm.at[j, pl.ds(off, _CHUNK)], c_v)
    for i in range(_CHUNK // _NLANE):
        sl = pl.ds(i * _NLANE, _NLANE)
        o_v[sl] = a_v[sl] + b_v[sl] + c_v[sl]
    pltpu.sync_copy(o_v, out_hbm.at[j, pl.ds(off, _CHUNK)])


def _sc_table(emb2, emb4, emb8):
    k = pl.kernel(
        _sc_table_body,
        out_type=jax.ShapeDtypeStruct((PERIOD, D_MODEL), jnp.float32),
        mesh=plsc.VectorSubcoreMesh(core_axis_name="c", subcore_axis_name="s"),
        scratch_types=[
            pltpu.VMEM((_CHUNK,), jnp.float32),
            pltpu.VMEM((_CHUNK,), jnp.float32),
            pltpu.VMEM((_CHUNK,), jnp.float32),
            pltpu.VMEM((_CHUNK,), jnp.float32),
        ],
    )
    return k(emb2, emb4, emb8)


def _tc_body(x_ref, t_ref, o_ref):
    add = jnp.tile(t_ref[...], (ROWS_PER_BLOCK // PERIOD, 1, 1))
    o_ref[...] = x_ref[...] + add


def kernel(x, emb2, emb4, emb8):
    table = _sc_table(emb2, emb4, emb8)
    L = x.shape[0]
    grid = (L // ROWS_PER_BLOCK,)
    return pl.pallas_call(
        _tc_body,
        grid=grid,
        in_specs=[
            pl.BlockSpec((ROWS_PER_BLOCK, SEQ, D_MODEL), lambda i: (i, 0, 0)),
            pl.BlockSpec((PERIOD, 1, D_MODEL), lambda i: (0, 0, 0)),
        ],
        out_specs=pl.BlockSpec((ROWS_PER_BLOCK, SEQ, D_MODEL), lambda i: (i, 0, 0)),
        out_shape=jax.ShapeDtypeStruct((L, SEQ, D_MODEL), x.dtype),
        compiler_params=pltpu.CompilerParams(
            dimension_semantics=("parallel",),
        ),
    )(x, table[:, None, :])


# speedup vs baseline: 1.0131x; 1.0131x over previous
import jax
import jax.numpy as jnp
from jax import lax
from jax.experimental import pallas as pl
from jax.experimental.pallas import tpu as pltpu
from jax.experimental.pallas import tpu_sc as plsc

D_MODEL = 1024
SEQ = 4
PERIOD = 8
ROWS_PER_BLOCK = 256
_CHUNK = 256
_NLANE = 16


def _sc_table_body(e2_hbm, e4_hbm, e8_hbm, out_hbm, a_v, b_v, c_v, o_v):
    c = lax.axis_index("c")
    s = lax.axis_index("s")
    w = s * 2 + c
    j = w % PERIOD
    off = (w // PERIOD) * _CHUNK
    pltpu.sync_copy(e2_hbm.at[j % 2, pl.ds(off, _CHUNK)], a_v)
    pltpu.sync_copy(e4_hbm.at[j % 4, pl.ds(off, _CHUNK)], b_v)
    pltpu.sync_copy(e8_hbm.at[j, pl.ds(off, _CHUNK)], c_v)
    for i in range(_CHUNK // _NLANE):
        sl = pl.ds(i * _NLANE, _NLANE)
        o_v[sl] = a_v[sl] + b_v[sl] + c_v[sl]
    pltpu.sync_copy(o_v, out_hbm.at[j, 0, pl.ds(off, _CHUNK)])


def _sc_table(emb2, emb4, emb8):
    k = pl.kernel(
        _sc_table_body,
        out_type=jax.ShapeDtypeStruct((PERIOD, 1, D_MODEL), jnp.float32),
        mesh=plsc.VectorSubcoreMesh(core_axis_name="c", subcore_axis_name="s"),
        scratch_types=[
            pltpu.VMEM((_CHUNK,), jnp.float32),
            pltpu.VMEM((_CHUNK,), jnp.float32),
            pltpu.VMEM((_CHUNK,), jnp.float32),
            pltpu.VMEM((_CHUNK,), jnp.float32),
        ],
    )
    return k(emb2, emb4, emb8)


def _tc_body(x_ref, t_ref, o_ref):
    add = jnp.tile(t_ref[...], (ROWS_PER_BLOCK // PERIOD, 1, 1))
    o_ref[...] = x_ref[...] + add


def kernel(x, emb2, emb4, emb8):
    table = _sc_table(emb2, emb4, emb8)
    L = x.shape[0]
    grid = (L // ROWS_PER_BLOCK,)
    return pl.pallas_call(
        _tc_body,
        grid=grid,
        in_specs=[
            pl.BlockSpec((ROWS_PER_BLOCK, SEQ, D_MODEL), lambda i: (i, 0, 0)),
            pl.BlockSpec((PERIOD, 1, D_MODEL), lambda i: (0, 0, 0)),
        ],
        out_specs=pl.BlockSpec((ROWS_PER_BLOCK, SEQ, D_MODEL), lambda i: (i, 0, 0)),
        out_shape=jax.ShapeDtypeStruct((L, SEQ, D_MODEL), x.dtype),
        compiler_params=pltpu.CompilerParams(
            dimension_semantics=("parallel",),
        ),
    )(x, table)


# speedup vs baseline: 1.0312x; 1.0178x over previous
import jax
import jax.numpy as jnp
from jax import lax
from jax.experimental import pallas as pl
from jax.experimental.pallas import tpu as pltpu
from jax.experimental.pallas import tpu_sc as plsc

D_MODEL = 1024
SEQ = 4
PERIOD = 8
ROWS_PER_BLOCK = 512
_CHUNK = 256
_NLANE = 16


def _sc_table_body(e2_hbm, e4_hbm, e8_hbm, out_hbm, a_v, b_v, c_v, o_v):
    c = lax.axis_index("c")
    s = lax.axis_index("s")
    w = s * 2 + c
    j = w % PERIOD
    off = (w // PERIOD) * _CHUNK
    pltpu.sync_copy(e2_hbm.at[j % 2, pl.ds(off, _CHUNK)], a_v)
    pltpu.sync_copy(e4_hbm.at[j % 4, pl.ds(off, _CHUNK)], b_v)
    pltpu.sync_copy(e8_hbm.at[j, pl.ds(off, _CHUNK)], c_v)
    for i in range(_CHUNK // _NLANE):
        sl = pl.ds(i * _NLANE, _NLANE)
        o_v[sl] = a_v[sl] + b_v[sl] + c_v[sl]
    pltpu.sync_copy(o_v, out_hbm.at[j, 0, pl.ds(off, _CHUNK)])


def _sc_table(emb2, emb4, emb8):
    k = pl.kernel(
        _sc_table_body,
        out_type=jax.ShapeDtypeStruct((PERIOD, 1, D_MODEL), jnp.float32),
        mesh=plsc.VectorSubcoreMesh(core_axis_name="c", subcore_axis_name="s"),
        scratch_types=[
            pltpu.VMEM((_CHUNK,), jnp.float32),
            pltpu.VMEM((_CHUNK,), jnp.float32),
            pltpu.VMEM((_CHUNK,), jnp.float32),
            pltpu.VMEM((_CHUNK,), jnp.float32),
        ],
    )
    return k(emb2, emb4, emb8)


def _tc_body(x_ref, t_ref, o_ref):
    add = jnp.tile(t_ref[...], (ROWS_PER_BLOCK // PERIOD, 1, 1))
    o_ref[...] = x_ref[...] + add


def kernel(x, emb2, emb4, emb8):
    table = _sc_table(emb2, emb4, emb8)
    L = x.shape[0]
    grid = (L // ROWS_PER_BLOCK,)
    return pl.pallas_call(
        _tc_body,
        grid=grid,
        in_specs=[
            pl.BlockSpec((ROWS_PER_BLOCK, SEQ, D_MODEL), lambda i: (i, 0, 0)),
            pl.BlockSpec((PERIOD, 1, D_MODEL), lambda i: (0, 0, 0)),
        ],
        out_specs=pl.BlockSpec((ROWS_PER_BLOCK, SEQ, D_MODEL), lambda i: (i, 0, 0)),
        out_shape=jax.ShapeDtypeStruct((L, SEQ, D_MODEL), x.dtype),
        compiler_params=pltpu.CompilerParams(
            dimension_semantics=("parallel",),
        ),
    )(x, table)


# speedup vs baseline: 1.0436x; 1.0121x over previous
import jax
import jax.numpy as jnp
from jax import lax
from jax.experimental import pallas as pl
from jax.experimental.pallas import tpu as pltpu
from jax.experimental.pallas import tpu_sc as plsc

D_MODEL = 1024
SEQ = 4
PERIOD = 8
ROWS_PER_BLOCK = 512
_CHUNK = 512
_NLANE = 16


def _sc_table_body(e2_hbm, e4_hbm, e8_hbm, out_hbm, a_v, b_v, c_v, o_v):
    w = lax.axis_index("s")
    j = w % PERIOD
    off = (w // PERIOD) * _CHUNK
    pltpu.sync_copy(e2_hbm.at[j % 2, pl.ds(off, _CHUNK)], a_v)
    pltpu.sync_copy(e4_hbm.at[j % 4, pl.ds(off, _CHUNK)], b_v)
    pltpu.sync_copy(e8_hbm.at[j, pl.ds(off, _CHUNK)], c_v)
    for i in range(_CHUNK // _NLANE):
        sl = pl.ds(i * _NLANE, _NLANE)
        o_v[sl] = a_v[sl] + b_v[sl] + c_v[sl]
    pltpu.sync_copy(o_v, out_hbm.at[j, 0, pl.ds(off, _CHUNK)])


def _sc_table(emb2, emb4, emb8):
    k = pl.kernel(
        _sc_table_body,
        out_type=jax.ShapeDtypeStruct((PERIOD, 1, D_MODEL), jnp.float32),
        mesh=plsc.VectorSubcoreMesh(
            core_axis_name="c", subcore_axis_name="s", num_cores=1
        ),
        scratch_types=[
            pltpu.VMEM((_CHUNK,), jnp.float32),
            pltpu.VMEM((_CHUNK,), jnp.float32),
            pltpu.VMEM((_CHUNK,), jnp.float32),
            pltpu.VMEM((_CHUNK,), jnp.float32),
        ],
    )
    return k(emb2, emb4, emb8)


def _tc_body(x_ref, t_ref, o_ref):
    add = jnp.tile(t_ref[...], (ROWS_PER_BLOCK // PERIOD, 1, 1))
    o_ref[...] = x_ref[...] + add


def kernel(x, emb2, emb4, emb8):
    table = _sc_table(emb2, emb4, emb8)
    L = x.shape[0]
    grid = (L // ROWS_PER_BLOCK,)
    return pl.pallas_call(
        _tc_body,
        grid=grid,
        in_specs=[
            pl.BlockSpec((ROWS_PER_BLOCK, SEQ, D_MODEL), lambda i: (i, 0, 0)),
            pl.BlockSpec((PERIOD, 1, D_MODEL), lambda i: (0, 0, 0)),
        ],
        out_specs=pl.BlockSpec((ROWS_PER_BLOCK, SEQ, D_MODEL), lambda i: (i, 0, 0)),
        out_shape=jax.ShapeDtypeStruct((L, SEQ, D_MODEL), x.dtype),
        compiler_params=pltpu.CompilerParams(
            dimension_semantics=("parallel",),
        ),
    )(x, table)


# speedup vs baseline: 1.0542x; 1.0102x over previous
import jax
import jax.numpy as jnp
from jax import lax
from jax.experimental import pallas as pl
from jax.experimental.pallas import tpu as pltpu
from jax.experimental.pallas import tpu_sc as plsc

D_MODEL = 1024
SEQ = 4
PERIOD = 8
ROWS_PER_BLOCK = 512
_CHUNK = 512
_NLANE = 16


def _sc_table_body(e2_hbm, e4_hbm, e8_hbm, out_hbm, a_v, b_v, c_v, o_v, sem):
    w = lax.axis_index("s")
    j = w % PERIOD
    off = (w // PERIOD) * _CHUNK
    cp_a = pltpu.async_copy(e2_hbm.at[j % 2, pl.ds(off, _CHUNK)], a_v, sem)
    cp_b = pltpu.async_copy(e4_hbm.at[j % 4, pl.ds(off, _CHUNK)], b_v, sem)
    cp_c = pltpu.async_copy(e8_hbm.at[j, pl.ds(off, _CHUNK)], c_v, sem)
    cp_a.wait()
    cp_b.wait()
    cp_c.wait()
    for i in range(_CHUNK // _NLANE):
        sl = pl.ds(i * _NLANE, _NLANE)
        o_v[sl] = a_v[sl] + b_v[sl] + c_v[sl]
    pltpu.sync_copy(o_v, out_hbm.at[j, 0, pl.ds(off, _CHUNK)])


def _sc_table(emb2, emb4, emb8):
    k = pl.kernel(
        _sc_table_body,
        out_type=jax.ShapeDtypeStruct((PERIOD, 1, D_MODEL), jnp.float32),
        mesh=plsc.VectorSubcoreMesh(
            core_axis_name="c", subcore_axis_name="s", num_cores=1
        ),
        scratch_types=[
            pltpu.VMEM((_CHUNK,), jnp.float32),
            pltpu.VMEM((_CHUNK,), jnp.float32),
            pltpu.VMEM((_CHUNK,), jnp.float32),
            pltpu.VMEM((_CHUNK,), jnp.float32),
            pltpu.SemaphoreType.DMA,
        ],
    )
    return k(emb2, emb4, emb8)


def _tc_body(x_ref, t_ref, o_ref):
    add = jnp.tile(t_ref[...], (ROWS_PER_BLOCK // PERIOD, 1, 1))
    o_ref[...] = x_ref[...] + add


def kernel(x, emb2, emb4, emb8):
    table = _sc_table(emb2, emb4, emb8)
    L = x.shape[0]
    grid = (L // ROWS_PER_BLOCK,)
    return pl.pallas_call(
        _tc_body,
        grid=grid,
        in_specs=[
            pl.BlockSpec((ROWS_PER_BLOCK, SEQ, D_MODEL), lambda i: (i, 0, 0)),
            pl.BlockSpec((PERIOD, 1, D_MODEL), lambda i: (0, 0, 0)),
        ],
        out_specs=pl.BlockSpec((ROWS_PER_BLOCK, SEQ, D_MODEL), lambda i: (i, 0, 0)),
        out_shape=jax.ShapeDtypeStruct((L, SEQ, D_MODEL), x.dtype),
        compiler_params=pltpu.CompilerParams(
            dimension_semantics=("parallel",),
        ),
    )(x, table)
